# baseline (speedup 1.0000x reference)
import functools

import jax
import jax.numpy as jnp
from jax import lax
from jax.experimental import pallas as pl
from jax.experimental.pallas import tpu as pltpu
from jax.experimental.pallas import tpu_sc as plsc

B, C, H, W = 4, 96, 384, 384
HB = 16
NC, NS, L = 2, 16, 16
CP = 128
NW = NC * NS
ROWS = H * W
P = 128
HP = 64
WP = W // P
TOT_CHUNKS = ROWS // P
N_CHUNKS = TOT_CHUNKS // NW
NHALF = 2 * N_CHUNKS
CPB = HB * WP


def _prep_body(x_ref, flow_ref, xt_ref, rec_ref, fl_ref, chm_ref):
    hi = pl.program_id(0)
    xb = x_ref[0].reshape(C, HB * W)
    xt_ref[...] = jnp.concatenate(
        [xb.T, jnp.zeros((HB * W, CP - C), jnp.float32)], axis=1)

    fxv = flow_ref[0, 0]
    fyv = flow_ref[0, 1]
    wv = lax.broadcasted_iota(jnp.int32, (HB, W), 1).astype(jnp.float32)
    hv = (hi * HB + lax.broadcasted_iota(jnp.int32, (HB, W), 0)
          ).astype(jnp.float32)
    vx = 2.0 * (wv + fxv) / float(W - 2)
    vy = 2.0 * (hv + fyv) / float(H - 2)
    ix = ((vx + 1.0) * W - 1.0) / 2.0
    iy = ((vy + 1.0) * H - 1.0) / 2.0
    ix0 = jnp.floor(ix)
    iy0 = jnp.floor(iy)
    wx1 = ix - ix0
    wx0 = 1.0 - wx1
    wy1 = iy - iy0
    wy0 = 1.0 - wy1
    vx0 = (ix0 >= 0.0) & (ix0 <= W - 1.0)
    vx1 = (ix0 + 1.0 >= 0.0) & (ix0 + 1.0 <= W - 1.0)
    vy0 = (iy0 >= 0.0) & (iy0 <= H - 1.0)
    vy1 = (iy0 + 1.0 >= 0.0) & (iy0 + 1.0 <= H - 1.0)
    f32 = jnp.float32
    m = ((vy0 & vx0).astype(f32) * (wy0 * wx0)
         + (vy0 & vx1).astype(f32) * (wy0 * wx1)
         + (vy1 & vx0).astype(f32) * (wy1 * wx0)
         + (vy1 & vx1).astype(f32) * (wy1 * wx1))
    fmask = jnp.where(m < 0.9999, 0.0, 1.0)

    ix0c = jnp.clip(ix0, -2.0, float(W + 1)).astype(jnp.int32)
    iy0c = jnp.clip(iy0, -2.0, float(H + 1)).astype(jnp.int32)
    cx = jnp.clip(ix0c, 0, W - 2)
    ry0 = jnp.clip(iy0c, 0, H - 1)
    ry1 = jnp.clip(iy0c + 1, 0, H - 1)
    wx0v = jnp.where(vx0, wx0, 0.0)
    wx1v = jnp.where(vx1, wx1, 0.0)
    cwA = (jnp.where(ix0c == cx, wx0v, 0.0)
           + jnp.where(ix0c + 1 == cx, wx1v, 0.0))
    cwB = (jnp.where(ix0c == cx + 1, wx0v, 0.0)
           + jnp.where(ix0c + 1 == cx + 1, wx1v, 0.0))
    rw0 = jnp.where(vy0, wy0, 0.0) * fmask
    rw1 = jnp.where(vy1, wy1, 0.0) * fmask

    ia = ry0 * W + cx
    ic = ry1 * W + cx
    qA = rw0 * cwA
    qB = rw0 * cwB
    qC = rw1 * cwA
    qD = rw1 * cwB

    bc = lambda a: a.astype(jnp.float32)
    z = jnp.zeros((HB, W), jnp.float32)
    rP = lambda a: a.reshape(HB, WP, P).reshape(CPB, P)
    rec_ref[...] = jnp.stack(
        [rP(bc(ia)), rP(bc(ic)), rP(qA), rP(qB), rP(qC), rP(qD),
         rP(z), rP(z)], axis=1)

    am = jnp.maximum(jnp.maximum(jnp.abs(qA), jnp.abs(qB)),
                     jnp.maximum(jnp.abs(qC), jnp.abs(qD)))
    am3 = jnp.max(am.reshape(HB, WP, P), axis=2)
    fl = (am3 > 0.0).astype(jnp.int32)
    fl_ref[...] = jnp.broadcast_to(
        fl[:, :, None], (HB, WP, 16)).reshape(HB, WP * 16)
    chm_ref[...] = jnp.broadcast_to(
        fl[:, :, None].astype(jnp.float32), (HB, WP, P)).reshape(HB, W)


_row_spec = pl.BlockSpec((HB, W), lambda i: (i, 0))

_prep_call = pl.pallas_call(
    _prep_body,
    grid=(H // HB,),
    in_specs=[
        pl.BlockSpec((1, C, HB, W), lambda i: (0, 0, i, 0)),
        pl.BlockSpec((1, 2, HB, W), lambda i: (0, 0, i, 0)),
    ],
    out_specs=[
        pl.BlockSpec((HB * W, CP), lambda i: (i, 0)),
        pl.BlockSpec((CPB, 8, CP), lambda i: (i, 0, 0)),
        pl.BlockSpec((HB, WP * 16), lambda i: (i, 0)),
        _row_spec,
    ],
    out_shape=[
        jax.ShapeDtypeStruct((ROWS, CP), jnp.float32),
        jax.ShapeDtypeStruct((TOT_CHUNKS, 8, CP), jnp.float32),
        jax.ShapeDtypeStruct((H, WP * 16), jnp.int32),
        jax.ShapeDtypeStruct((H, W), jnp.float32),
    ],
)


def _finish_body(ot_ref, chm_ref, y_ref):
    y = ot_ref[...][:, :C].T.reshape(C, HB, W)
    y_ref[0] = jnp.where(chm_ref[...][None] > 0.0, y, 0.0)


_finish_call = pl.pallas_call(
    _finish_body,
    grid=(H // HB,),
    in_specs=[
        pl.BlockSpec((HB * W, CP), lambda i: (i, 0)),
        _row_spec,
    ],
    out_specs=pl.BlockSpec((1, C, HB, W), lambda i: (0, 0, i, 0)),
    out_shape=jax.ShapeDtypeStruct((1, C, H, W), jnp.float32),
)


@functools.cache
def _make_sc_warp():
  kern = functools.partial(
    pl.kernel,
    out_type=jax.ShapeDtypeStruct((ROWS, CP), jnp.float32),
    mesh=plsc.VectorSubcoreMesh(core_axis_name="c", subcore_axis_name="s"),
    scratch_types=[
        pltpu.VMEM((N_CHUNKS, 16), jnp.int32),
        pltpu.VMEM((8, CP), jnp.float32),
        pltpu.VMEM((8, CP), jnp.float32),
        pltpu.VMEM((HP,), jnp.int32),
        pltpu.VMEM((HP,), jnp.int32),
        pltpu.VMEM((HP,), jnp.int32),
        pltpu.VMEM((HP,), jnp.int32),
        pltpu.VMEM((HP,), jnp.int32),
        pltpu.VMEM((HP,), jnp.int32),
        pltpu.VMEM((HP,), jnp.int32),
        pltpu.VMEM((HP,), jnp.int32),
        pltpu.VMEM((HP, CP), jnp.float32),
        pltpu.VMEM((HP, CP), jnp.float32),
        pltpu.VMEM((HP, CP), jnp.float32),
        pltpu.VMEM((HP, CP), jnp.float32),
        pltpu.VMEM((HP, CP), jnp.float32),
        pltpu.VMEM((HP, CP), jnp.float32),
        pltpu.VMEM((HP, CP), jnp.float32),
        pltpu.VMEM((HP, CP), jnp.float32),
        pltpu.SemaphoreType.DMA,
        pltpu.SemaphoreType.DMA,
    ],
  )

  @kern
  def _sc_warp(xt_hbm, rec_hbm, flp_hbm, out_hbm,
               flv, recv0, recv1,
               iaA0, iaB0, iaC0, iaD0, iaA1, iaB1, iaC1, iaD1,
               dstA0, dstB0, dstC0, dstD0, dstA1, dstB1, dstC1, dstD1,
               sem0, sem1):
    wid = lax.axis_index("s") * NC + lax.axis_index("c")
    sets = ((recv0, iaA0, iaB0, iaC0, iaD0, dstA0, dstB0, dstC0, dstD0,
             sem0),
            (recv1, iaA1, iaB1, iaC1, iaD1, dstA1, dstB1, dstC1, dstD1,
             sem1))

    pltpu.sync_copy(flp_hbm.at[wid], flv)

    def issue(s, cid, hoff):
        recv, iA, iB, iC, iD, dA, dB, dC, dD, sem = sets[s]
        pltpu.sync_copy(rec_hbm.at[cid], recv)
        for t in range(HP // L):
            sl = pl.ds(t * L, L)
            slr = pl.ds(hoff + t * L, L)
            a = recv[0, slr].astype(jnp.int32)
            c = recv[1, slr].astype(jnp.int32)
            iA[sl] = a
            iB[sl] = a + 1
            iC[sl] = c
            iD[sl] = c + 1
        pltpu.async_copy(xt_hbm.at[iA], dA, sem)
        pltpu.async_copy(xt_hbm.at[iB], dB, sem)
        pltpu.async_copy(xt_hbm.at[iC], dC, sem)
        pltpu.async_copy(xt_hbm.at[iD], dD, sem)

    def consume(s, pcid, ph):
        recv, iA, iB, iC, iD, dA, dB, dC, dD, sem = sets[s]
        poff = ph * HP
        pltpu.make_async_copy(xt_hbm.at[iA], dA, sem).wait()
        pltpu.make_async_copy(xt_hbm.at[iB], dB, sem).wait()
        pltpu.make_async_copy(xt_hbm.at[iC], dC, sem).wait()
        pltpu.make_async_copy(xt_hbm.at[iD], dD, sem).wait()

        def grp_body(g, carry2):
            gsl = pl.ds(poff + g * L, L)
            wa16 = recv[2, gsl]
            wb16 = recv[3, gsl]
            wc16 = recv[4, gsl]
            wd16 = recv[5, gsl]
            dnums = lax.GatherDimensionNumbers(
                offset_dims=(), collapsed_slice_dims=(0,),
                start_index_map=(0,))
            for j in range(L):
                jv = jnp.full((L, 1), j, dtype=jnp.int32)
                bcast = lambda v: lax.gather(
                    v, jv, dnums, (1,), indices_are_sorted=True,
                    unique_indices=False,
                    mode=lax.GatherScatterMode.PROMISE_IN_BOUNDS)
                qa = bcast(wa16)
                qb = bcast(wb16)
                qc = bcast(wc16)
                qd = bcast(wd16)
                k = g * L + j
                for c0 in range(C // L):
                    sl = pl.ds(c0 * L, L)
                    dA[k, sl] = (qa * dA[k, sl] + qb * dB[k, sl]
                                 + qc * dC[k, sl] + qd * dD[k, sl])
            return carry2

        lax.fori_loop(0, HP // L, grp_body, 0)
        pltpu.sync_copy(dA, out_hbm.at[pl.ds(pcid * P + poff, HP)])

    def chunk_body(j, carry):
        pend, pcid, ph, par = carry
        is_last = j >= NHALF
        jc = jnp.minimum(j, NHALF - 1)
        rc = jc // 2
        h = jc - rc * 2
        act = jnp.where(is_last, 0, flv[rc, pl.ds(0, L)][0])
        cid = rc * NW + wid
        do_issue = act > 0
        do_consume = (pend > 0) & (do_issue | is_last)
        pl.when(do_issue & (par == 1))(lambda: issue(0, cid, h * HP))
        pl.when(do_issue & (par == 0))(lambda: issue(1, cid, h * HP))
        pl.when(do_consume & (par == 0))(lambda: consume(0, pcid, ph))
        pl.when(do_consume & (par == 1))(lambda: consume(1, pcid, ph))
        pend = jnp.where(do_issue, 1, jnp.where(is_last, 0, pend))
        pcid = jnp.where(do_issue, cid, pcid)
        ph = jnp.where(do_issue, h, ph)
        par = jnp.where(do_issue, 1 - par, par)
        return pend, pcid, ph, par

    lax.fori_loop(0, NHALF + 1, chunk_body,
                  (jnp.int32(0), jnp.int32(0), jnp.int32(0), jnp.int32(1)))

  return _sc_warp


def kernel(x, flow):
    sc = _make_sc_warp()
    outs = []
    for b in range(B):
        x_t, rec, fl, chm = _prep_call(x[b:b + 1], flow[b:b + 1])
        flp = (fl.reshape(TOT_CHUNKS, 16)
                 .reshape(N_CHUNKS, NW, 16)
                 .transpose(1, 0, 2))
        outs.append(_finish_call(sc(x_t, rec, flp), chm))
    return jnp.concatenate(outs, axis=0)


# speedup vs baseline: 1.0145x; 1.0145x over previous
import functools

import jax
import jax.numpy as jnp
from jax import lax
from jax.experimental import pallas as pl
from jax.experimental.pallas import tpu as pltpu
from jax.experimental.pallas import tpu_sc as plsc

B, C, H, W = 4, 96, 384, 384
HB = 16
NC, NS, L = 2, 16, 16
CP = 128
NW = NC * NS
BHW = B * H * W
GB = 2
NG = B // GB
GHW = GB * H * W
P = 128
HP = 64
WP = W // P
TOT_CHUNKS = GHW // P
N_CHUNKS = TOT_CHUNKS // NW
NHALF = 2 * N_CHUNKS
CPB = HB * WP


def _prep_body(x_ref, flow_ref, xt_ref, rec_ref, fl_ref, chm_ref):
    bb = pl.program_id(0)
    hi = pl.program_id(1)
    xb = x_ref[0].reshape(C, HB * W)
    xt_ref[...] = jnp.concatenate(
        [xb.T, jnp.zeros((HB * W, CP - C), jnp.float32)], axis=1)

    fxv = flow_ref[0, 0]
    fyv = flow_ref[0, 1]
    wv = lax.broadcasted_iota(jnp.int32, (HB, W), 1).astype(jnp.float32)
    hv = (hi * HB + lax.broadcasted_iota(jnp.int32, (HB, W), 0)
          ).astype(jnp.float32)
    vx = 2.0 * (wv + fxv) / float(W - 2)
    vy = 2.0 * (hv + fyv) / float(H - 2)
    ix = ((vx + 1.0) * W - 1.0) / 2.0
    iy = ((vy + 1.0) * H - 1.0) / 2.0
    ix0 = jnp.floor(ix)
    iy0 = jnp.floor(iy)
    wx1 = ix - ix0
    wx0 = 1.0 - wx1
    wy1 = iy - iy0
    wy0 = 1.0 - wy1
    vx0 = (ix0 >= 0.0) & (ix0 <= W - 1.0)
    vx1 = (ix0 + 1.0 >= 0.0) & (ix0 + 1.0 <= W - 1.0)
    vy0 = (iy0 >= 0.0) & (iy0 <= H - 1.0)
    vy1 = (iy0 + 1.0 >= 0.0) & (iy0 + 1.0 <= H - 1.0)
    f32 = jnp.float32
    m = ((vy0 & vx0).astype(f32) * (wy0 * wx0)
         + (vy0 & vx1).astype(f32) * (wy0 * wx1)
         + (vy1 & vx0).astype(f32) * (wy1 * wx0)
         + (vy1 & vx1).astype(f32) * (wy1 * wx1))
    fmask = jnp.where(m < 0.9999, 0.0, 1.0)

    ix0c = jnp.clip(ix0, -2.0, float(W + 1)).astype(jnp.int32)
    iy0c = jnp.clip(iy0, -2.0, float(H + 1)).astype(jnp.int32)
    cx = jnp.clip(ix0c, 0, W - 2)
    ry0 = jnp.clip(iy0c, 0, H - 1)
    ry1 = jnp.clip(iy0c + 1, 0, H - 1)
    wx0v = jnp.where(vx0, wx0, 0.0)
    wx1v = jnp.where(vx1, wx1, 0.0)
    cwA = (jnp.where(ix0c == cx, wx0v, 0.0)
           + jnp.where(ix0c + 1 == cx, wx1v, 0.0))
    cwB = (jnp.where(ix0c == cx + 1, wx0v, 0.0)
           + jnp.where(ix0c + 1 == cx + 1, wx1v, 0.0))
    rw0 = jnp.where(vy0, wy0, 0.0) * fmask
    rw1 = jnp.where(vy1, wy1, 0.0) * fmask

    ia = (bb * H + ry0) * W + cx
    ic = (bb * H + ry1) * W + cx
    qA = rw0 * cwA
    qB = rw0 * cwB
    qC = rw1 * cwA
    qD = rw1 * cwB

    bc = lambda a: a.astype(jnp.float32)
    z = jnp.zeros((HB, W), jnp.float32)
    rP = lambda a: a.reshape(HB, WP, P).reshape(CPB, P)
    rec_ref[...] = jnp.stack(
        [rP(bc(ia)), rP(bc(ic)), rP(qA), rP(qB), rP(qC), rP(qD),
         rP(z), rP(z)], axis=1)

    am = jnp.maximum(jnp.maximum(jnp.abs(qA), jnp.abs(qB)),
                     jnp.maximum(jnp.abs(qC), jnp.abs(qD)))
    am3 = jnp.max(am.reshape(HB, WP, P), axis=2)
    fl = (am3 > 0.0).astype(jnp.int32)
    fl_ref[...] = jnp.broadcast_to(
        fl[:, :, None], (HB, WP, 16)).reshape(HB, WP * 16)
    chm_ref[...] = jnp.broadcast_to(
        fl[:, :, None].astype(jnp.float32), (HB, WP, P)).reshape(HB, W)


_row_spec = pl.BlockSpec((HB, W), lambda b, i: (b * (H // HB) + i, 0))

_prep_call = pl.pallas_call(
    _prep_body,
    grid=(GB, H // HB),
    in_specs=[
        pl.BlockSpec((1, C, HB, W), lambda b, i: (b, 0, i, 0)),
        pl.BlockSpec((1, 2, HB, W), lambda b, i: (b, 0, i, 0)),
    ],
    out_specs=[
        pl.BlockSpec((HB * W, CP), lambda b, i: (b * (H // HB) + i, 0)),
        pl.BlockSpec((CPB, 8, CP), lambda b, i: (b * (H // HB) + i, 0, 0)),
        pl.BlockSpec((HB, WP * 16), lambda b, i: (b * (H // HB) + i, 0)),
        _row_spec,
    ],
    out_shape=[
        jax.ShapeDtypeStruct((GHW, CP), jnp.float32),
        jax.ShapeDtypeStruct((TOT_CHUNKS, 8, CP), jnp.float32),
        jax.ShapeDtypeStruct((GB * H, WP * 16), jnp.int32),
        jax.ShapeDtypeStruct((GB * H, W), jnp.float32),
    ],
)


def _finish_body(ot_ref, chm_ref, y_ref):
    y = ot_ref[...][:, :C].T.reshape(C, HB, W)
    y_ref[0] = jnp.where(chm_ref[...][None] > 0.0, y, 0.0)


_finish_call = pl.pallas_call(
    _finish_body,
    grid=(GB, H // HB),
    in_specs=[
        pl.BlockSpec((HB * W, CP), lambda b, i: (b * (H // HB) + i, 0)),
        _row_spec,
    ],
    out_specs=pl.BlockSpec((1, C, HB, W), lambda b, i: (b, 0, i, 0)),
    out_shape=jax.ShapeDtypeStruct((GB, C, H, W), jnp.float32),
)


@functools.cache
def _make_sc_warp():
  kern = functools.partial(
    pl.kernel,
    out_type=jax.ShapeDtypeStruct((GHW, CP), jnp.float32),
    mesh=plsc.VectorSubcoreMesh(core_axis_name="c", subcore_axis_name="s"),
    scratch_types=[
        pltpu.VMEM((N_CHUNKS, 16), jnp.int32),
        pltpu.VMEM((8, CP), jnp.float32),
        pltpu.VMEM((8, CP), jnp.float32),
        pltpu.VMEM((HP,), jnp.int32),
        pltpu.VMEM((HP,), jnp.int32),
        pltpu.VMEM((HP,), jnp.int32),
        pltpu.VMEM((HP,), jnp.int32),
        pltpu.VMEM((HP,), jnp.int32),
        pltpu.VMEM((HP,), jnp.int32),
        pltpu.VMEM((HP,), jnp.int32),
        pltpu.VMEM((HP,), jnp.int32),
        pltpu.VMEM((HP, CP), jnp.float32),
        pltpu.VMEM((HP, CP), jnp.float32),
        pltpu.VMEM((HP, CP), jnp.float32),
        pltpu.VMEM((HP, CP), jnp.float32),
        pltpu.VMEM((HP, CP), jnp.float32),
        pltpu.VMEM((HP, CP), jnp.float32),
        pltpu.VMEM((HP, CP), jnp.float32),
        pltpu.VMEM((HP, CP), jnp.float32),
        pltpu.SemaphoreType.DMA,
        pltpu.SemaphoreType.DMA,
    ],
  )

  @kern
  def _sc_warp(xt_hbm, rec_hbm, flp_hbm, out_hbm,
               flv, recv0, recv1,
               iaA0, iaB0, iaC0, iaD0, iaA1, iaB1, iaC1, iaD1,
               dstA0, dstB0, dstC0, dstD0, dstA1, dstB1, dstC1, dstD1,
               sem0, sem1):
    wid = lax.axis_index("s") * NC + lax.axis_index("c")
    sets = ((recv0, iaA0, iaB0, iaC0, iaD0, dstA0, dstB0, dstC0, dstD0,
             sem0),
            (recv1, iaA1, iaB1, iaC1, iaD1, dstA1, dstB1, dstC1, dstD1,
             sem1))

    pltpu.sync_copy(flp_hbm.at[wid], flv)

    def issue(s, cid, hoff):
        recv, iA, iB, iC, iD, dA, dB, dC, dD, sem = sets[s]
        pltpu.sync_copy(rec_hbm.at[cid], recv)
        for t in range(HP // L):
            sl = pl.ds(t * L, L)
            slr = pl.ds(hoff + t * L, L)
            a = recv[0, slr].astype(jnp.int32)
            c = recv[1, slr].astype(jnp.int32)
            iA[sl] = a
            iB[sl] = a + 1
            iC[sl] = c
            iD[sl] = c + 1
        pltpu.async_copy(xt_hbm.at[iA], dA, sem)
        pltpu.async_copy(xt_hbm.at[iB], dB, sem)
        pltpu.async_copy(xt_hbm.at[iC], dC, sem)
        pltpu.async_copy(xt_hbm.at[iD], dD, sem)

    def consume(s, pcid, ph):
        recv, iA, iB, iC, iD, dA, dB, dC, dD, sem = sets[s]
        poff = ph * HP
        pltpu.make_async_copy(xt_hbm.at[iA], dA, sem).wait()
        pltpu.make_async_copy(xt_hbm.at[iB], dB, sem).wait()
        pltpu.make_async_copy(xt_hbm.at[iC], dC, sem).wait()
        pltpu.make_async_copy(xt_hbm.at[iD], dD, sem).wait()

        def grp_body(g, carry2):
            gsl = pl.ds(poff + g * L, L)
            wa16 = recv[2, gsl]
            wb16 = recv[3, gsl]
            wc16 = recv[4, gsl]
            wd16 = recv[5, gsl]
            dnums = lax.GatherDimensionNumbers(
                offset_dims=(), collapsed_slice_dims=(0,),
                start_index_map=(0,))
            for j in range(L):
                jv = jnp.full((L, 1), j, dtype=jnp.int32)
                bcast = lambda v: lax.gather(
                    v, jv, dnums, (1,), indices_are_sorted=True,
                    unique_indices=False,
                    mode=lax.GatherScatterMode.PROMISE_IN_BOUNDS)
                qa = bcast(wa16)
                qb = bcast(wb16)
                qc = bcast(wc16)
                qd = bcast(wd16)
                k = g * L + j
                for c0 in range(C // L):
                    sl = pl.ds(c0 * L, L)
                    dA[k, sl] = (qa * dA[k, sl] + qb * dB[k, sl]
                                 + qc * dC[k, sl] + qd * dD[k, sl])
            return carry2

        lax.fori_loop(0, HP // L, grp_body, 0)
        pltpu.sync_copy(dA, out_hbm.at[pl.ds(pcid * P + poff, HP)])

    def chunk_body(j, carry):
        pend, pcid, ph, par = carry
        is_last = j >= NHALF
        jc = jnp.minimum(j, NHALF - 1)
        rc = jc // 2
        h = jc - rc * 2
        act = jnp.where(is_last, 0, flv[rc, pl.ds(0, L)][0])
        cid = rc * NW + wid
        do_issue = act > 0
        do_consume = (pend > 0) & (do_issue | is_last)
        pl.when(do_issue & (par == 1))(lambda: issue(0, cid, h * HP))
        pl.when(do_issue & (par == 0))(lambda: issue(1, cid, h * HP))
        pl.when(do_consume & (par == 0))(lambda: consume(0, pcid, ph))
        pl.when(do_consume & (par == 1))(lambda: consume(1, pcid, ph))
        pend = jnp.where(do_issue, 1, jnp.where(is_last, 0, pend))
        pcid = jnp.where(do_issue, cid, pcid)
        ph = jnp.where(do_issue, h, ph)
        par = jnp.where(do_issue, 1 - par, par)
        return pend, pcid, ph, par

    lax.fori_loop(0, NHALF + 1, chunk_body,
                  (jnp.int32(0), jnp.int32(0), jnp.int32(0), jnp.int32(1)))

  return _sc_warp


def kernel(x, flow):
    sc = _make_sc_warp()
    outs = []
    for g in range(NG):
        sl = slice(g * GB, (g + 1) * GB)
        x_t, rec, fl, chm = _prep_call(x[sl], flow[sl])
        flp = (fl.reshape(TOT_CHUNKS, 16)
                 .reshape(N_CHUNKS, NW, 16)
                 .transpose(1, 0, 2))
        outs.append(_finish_call(sc(x_t, rec, flp), chm))
    return jnp.concatenate(outs, axis=0)


# speedup vs baseline: 1.2931x; 1.2746x over previous
import functools

import jax
import jax.numpy as jnp
from jax import lax
from jax.experimental import pallas as pl
from jax.experimental.pallas import tpu as pltpu
from jax.experimental.pallas import tpu_sc as plsc

B, C, H, W = 4, 96, 384, 384
HB = 16
NC, NS, L = 2, 16, 16
CP = 128
NW = NC * NS
BHW = B * H * W
P = 128
HP = 64
WP = W // P
TOT_CHUNKS = BHW // P
N_CHUNKS = TOT_CHUNKS // NW
NHALF = 2 * N_CHUNKS
CPB = HB * WP


def _prep_body(x_ref, flow_ref, xt_ref, rec_ref, fl_ref, chm_ref):
    bb = pl.program_id(0)
    hi = pl.program_id(1)
    xb = x_ref[0].reshape(C, HB * W)
    xt_ref[...] = jnp.concatenate(
        [xb.T, jnp.zeros((HB * W, CP - C), jnp.float32)], axis=1)

    fxv = flow_ref[0, 0]
    fyv = flow_ref[0, 1]
    wv = lax.broadcasted_iota(jnp.int32, (HB, W), 1).astype(jnp.float32)
    hv = (hi * HB + lax.broadcasted_iota(jnp.int32, (HB, W), 0)
          ).astype(jnp.float32)
    vx = 2.0 * (wv + fxv) / float(W - 2)
    vy = 2.0 * (hv + fyv) / float(H - 2)
    ix = ((vx + 1.0) * W - 1.0) / 2.0
    iy = ((vy + 1.0) * H - 1.0) / 2.0
    ix0 = jnp.floor(ix)
    iy0 = jnp.floor(iy)
    wx1 = ix - ix0
    wx0 = 1.0 - wx1
    wy1 = iy - iy0
    wy0 = 1.0 - wy1
    vx0 = (ix0 >= 0.0) & (ix0 <= W - 1.0)
    vx1 = (ix0 + 1.0 >= 0.0) & (ix0 + 1.0 <= W - 1.0)
    vy0 = (iy0 >= 0.0) & (iy0 <= H - 1.0)
    vy1 = (iy0 + 1.0 >= 0.0) & (iy0 + 1.0 <= H - 1.0)
    f32 = jnp.float32
    m = ((vy0 & vx0).astype(f32) * (wy0 * wx0)
         + (vy0 & vx1).astype(f32) * (wy0 * wx1)
         + (vy1 & vx0).astype(f32) * (wy1 * wx0)
         + (vy1 & vx1).astype(f32) * (wy1 * wx1))
    fmask = jnp.where(m < 0.9999, 0.0, 1.0)

    ix0c = jnp.clip(ix0, -2.0, float(W + 1)).astype(jnp.int32)
    iy0c = jnp.clip(iy0, -2.0, float(H + 1)).astype(jnp.int32)
    cx = jnp.clip(ix0c, 0, W - 2)
    ry0 = jnp.clip(iy0c, 0, H - 1)
    ry1 = jnp.clip(iy0c + 1, 0, H - 1)
    wx0v = jnp.where(vx0, wx0, 0.0)
    wx1v = jnp.where(vx1, wx1, 0.0)
    cwA = (jnp.where(ix0c == cx, wx0v, 0.0)
           + jnp.where(ix0c + 1 == cx, wx1v, 0.0))
    cwB = (jnp.where(ix0c == cx + 1, wx0v, 0.0)
           + jnp.where(ix0c + 1 == cx + 1, wx1v, 0.0))
    rw0 = jnp.where(vy0, wy0, 0.0) * fmask
    rw1 = jnp.where(vy1, wy1, 0.0) * fmask

    ia = (bb * H + ry0) * W + cx
    ic = (bb * H + ry1) * W + cx
    qA = rw0 * cwA
    qB = rw0 * cwB
    qC = rw1 * cwA
    qD = rw1 * cwB

    bc = lambda a: a.astype(jnp.float32)
    z = jnp.zeros((HB, W), jnp.float32)
    rP = lambda a: a.reshape(HB, WP, P).reshape(CPB, P)
    rec_ref[...] = jnp.stack(
        [rP(bc(ia)), rP(bc(ic)), rP(qA), rP(qB), rP(qC), rP(qD),
         rP(z), rP(z)], axis=1)

    am = jnp.maximum(jnp.maximum(jnp.abs(qA), jnp.abs(qB)),
                     jnp.maximum(jnp.abs(qC), jnp.abs(qD)))
    am3 = jnp.max(am.reshape(HB, WP, P), axis=2)
    fl = (am3 > 0.0).astype(jnp.int32)
    fl_ref[...] = jnp.broadcast_to(
        fl[:, :, None], (HB, WP, 16)).reshape(HB, WP * 16)
    chm_ref[...] = jnp.broadcast_to(
        fl[:, :, None].astype(jnp.float32), (HB, WP, P)).reshape(HB, W)


_row_spec = pl.BlockSpec((HB, W), lambda b, i: (b * (H // HB) + i, 0))

_prep_call = pl.pallas_call(
    _prep_body,
    grid=(B, H // HB),
    in_specs=[
        pl.BlockSpec((1, C, HB, W), lambda b, i: (b, 0, i, 0)),
        pl.BlockSpec((1, 2, HB, W), lambda b, i: (b, 0, i, 0)),
    ],
    out_specs=[
        pl.BlockSpec((HB * W, CP), lambda b, i: (b * (H // HB) + i, 0)),
        pl.BlockSpec((CPB, 8, CP), lambda b, i: (b * (H // HB) + i, 0, 0)),
        pl.BlockSpec((HB, WP * 16), lambda b, i: (b * (H // HB) + i, 0)),
        _row_spec,
    ],
    out_shape=[
        jax.ShapeDtypeStruct((BHW, CP), jnp.float32),
        jax.ShapeDtypeStruct((TOT_CHUNKS, 8, CP), jnp.float32),
        jax.ShapeDtypeStruct((B * H, WP * 16), jnp.int32),
        jax.ShapeDtypeStruct((B * H, W), jnp.float32),
    ],
)


def _finish_body(ot_ref, chm_ref, y_ref):
    y = ot_ref[...][:, :C].T.reshape(C, HB, W)
    y_ref[0] = jnp.where(chm_ref[...][None] > 0.0, y, 0.0)


_finish_call = pl.pallas_call(
    _finish_body,
    grid=(B, H // HB),
    in_specs=[
        pl.BlockSpec((HB * W, CP), lambda b, i: (b * (H // HB) + i, 0)),
        _row_spec,
    ],
    out_specs=pl.BlockSpec((1, C, HB, W), lambda b, i: (b, 0, i, 0)),
    out_shape=jax.ShapeDtypeStruct((B, C, H, W), jnp.float32),
)


@functools.cache
def _make_sc_warp():
  kern = functools.partial(
    pl.kernel,
    out_type=jax.ShapeDtypeStruct((BHW, CP), jnp.float32),
    mesh=plsc.VectorSubcoreMesh(core_axis_name="c", subcore_axis_name="s"),
    scratch_types=[
        pltpu.VMEM((N_CHUNKS, 16), jnp.int32),
        pltpu.VMEM((8, CP), jnp.float32),
        pltpu.VMEM((8, CP), jnp.float32),
        pltpu.VMEM((HP,), jnp.int32),
        pltpu.VMEM((HP,), jnp.int32),
        pltpu.VMEM((HP,), jnp.int32),
        pltpu.VMEM((HP,), jnp.int32),
        pltpu.VMEM((HP,), jnp.int32),
        pltpu.VMEM((HP,), jnp.int32),
        pltpu.VMEM((HP,), jnp.int32),
        pltpu.VMEM((HP,), jnp.int32),
        pltpu.VMEM((HP, CP), jnp.float32),
        pltpu.VMEM((HP, CP), jnp.float32),
        pltpu.VMEM((HP, CP), jnp.float32),
        pltpu.VMEM((HP, CP), jnp.float32),
        pltpu.VMEM((HP, CP), jnp.float32),
        pltpu.VMEM((HP, CP), jnp.float32),
        pltpu.VMEM((HP, CP), jnp.float32),
        pltpu.VMEM((HP, CP), jnp.float32),
        pltpu.SemaphoreType.DMA,
        pltpu.SemaphoreType.DMA,
    ],
  )

  @kern
  def _sc_warp(xt_hbm, rec_hbm, flp_hbm, out_hbm,
               flv, recv0, recv1,
               iaA0, iaB0, iaC0, iaD0, iaA1, iaB1, iaC1, iaD1,
               dstA0, dstB0, dstC0, dstD0, dstA1, dstB1, dstC1, dstD1,
               sem0, sem1):
    wid = lax.axis_index("s") * NC + lax.axis_index("c")
    sets = ((recv0, iaA0, iaB0, iaC0, iaD0, dstA0, dstB0, dstC0, dstD0,
             sem0),
            (recv1, iaA1, iaB1, iaC1, iaD1, dstA1, dstB1, dstC1, dstD1,
             sem1))

    pltpu.sync_copy(flp_hbm.at[wid], flv)

    def issue(s, cid, hoff):
        recv, iA, iB, iC, iD, dA, dB, dC, dD, sem = sets[s]
        pltpu.sync_copy(rec_hbm.at[cid], recv)
        for t in range(HP // L):
            sl = pl.ds(t * L, L)
            slr = pl.ds(hoff + t * L, L)
            a = recv[0, slr].astype(jnp.int32)
            c = recv[1, slr].astype(jnp.int32)
            iA[sl] = a
            iB[sl] = a + 1
            iC[sl] = c
            iD[sl] = c + 1
        pltpu.async_copy(xt_hbm.at[iA], dA, sem)
        pltpu.async_copy(xt_hbm.at[iB], dB, sem)
        pltpu.async_copy(xt_hbm.at[iC], dC, sem)
        pltpu.async_copy(xt_hbm.at[iD], dD, sem)

    def consume(s, pcid, ph):
        recv, iA, iB, iC, iD, dA, dB, dC, dD, sem = sets[s]
        poff = ph * HP
        pltpu.make_async_copy(xt_hbm.at[iA], dA, sem).wait()
        pltpu.make_async_copy(xt_hbm.at[iB], dB, sem).wait()
        pltpu.make_async_copy(xt_hbm.at[iC], dC, sem).wait()
        pltpu.make_async_copy(xt_hbm.at[iD], dD, sem).wait()

        def grp_body(g, carry2):
            gsl = pl.ds(poff + g * L, L)
            wa16 = recv[2, gsl]
            wb16 = recv[3, gsl]
            wc16 = recv[4, gsl]
            wd16 = recv[5, gsl]
            dnums = lax.GatherDimensionNumbers(
                offset_dims=(), collapsed_slice_dims=(0,),
                start_index_map=(0,))
            for j in range(L):
                jv = jnp.full((L, 1), j, dtype=jnp.int32)
                bcast = lambda v: lax.gather(
                    v, jv, dnums, (1,), indices_are_sorted=True,
                    unique_indices=False,
                    mode=lax.GatherScatterMode.PROMISE_IN_BOUNDS)
                qa = bcast(wa16)
                qb = bcast(wb16)
                qc = bcast(wc16)
                qd = bcast(wd16)
                k = g * L + j
                for c0 in range(C // L):
                    sl = pl.ds(c0 * L, L)
                    dA[k, sl] = (qa * dA[k, sl] + qb * dB[k, sl]
                                 + qc * dC[k, sl] + qd * dD[k, sl])
            return carry2

        lax.fori_loop(0, HP // L, grp_body, 0)
        pltpu.sync_copy(dA, out_hbm.at[pl.ds(pcid * P + poff, HP)])

    def chunk_body(j, carry):
        pend, pcid, ph, par = carry
        is_last = j >= NHALF
        jc = jnp.minimum(j, NHALF - 1)
        rc = jc // 2
        h = jc - rc * 2
        act = jnp.where(is_last, 0, flv[rc, pl.ds(0, L)][0])
        cid = rc * NW + wid
        do_issue = act > 0
        do_consume = (pend > 0) & (do_issue | is_last)
        pl.when(do_issue & (par == 1))(lambda: issue(0, cid, h * HP))
        pl.when(do_issue & (par == 0))(lambda: issue(1, cid, h * HP))
        pl.when(do_consume & (par == 0))(lambda: consume(0, pcid, ph))
        pl.when(do_consume & (par == 1))(lambda: consume(1, pcid, ph))
        pend = jnp.where(do_issue, 1, jnp.where(is_last, 0, pend))
        pcid = jnp.where(do_issue, cid, pcid)
        ph = jnp.where(do_issue, h, ph)
        par = jnp.where(do_issue, 1 - par, par)
        return pend, pcid, ph, par

    lax.fori_loop(0, NHALF + 1, chunk_body,
                  (jnp.int32(0), jnp.int32(0), jnp.int32(0), jnp.int32(1)))

  return _sc_warp


def kernel(x, flow):
    x_t, rec, fl, chm = _prep_call(x, flow)
    flp = (fl.reshape(TOT_CHUNKS, 16)
             .reshape(N_CHUNKS, NW, 16)
             .transpose(1, 0, 2))
    out_t = _make_sc_warp()(x_t, rec, flp)
    return _finish_call(out_t, chm)
